# baseline (speedup 1.0000x reference)
import functools

import jax
import jax.numpy as jnp
from jax import lax
from jax.experimental import pallas as pl
from jax.experimental.pallas import tpu as pltpu
from jax.experimental.pallas import tpu_sc as plsc

N = 4096
DIN = 384
D = 512
K = 6
KP = 8
BLK = 256
NBLK = N // BLK
OMIC_PAD = 1536
NEG = -1e30


def _leaky(x):
    return jnp.where(x > 0, x, 0.01 * x)


def _elu(x):
    return jnp.where(x > 0, x, jnp.exp(x) - 1.0)


def _omic_body(x_ref, w0_ref, b0_ref, w1_ref, b1_ref, o_ref):
    for i in range(6):
        h = lax.dot_general(x_ref[i:i + 1, :], w0_ref[i],
                            (((1,), (1,)), ((), ())),
                            preferred_element_type=jnp.float32)
        h = _elu(h + b0_ref[i:i + 1, :])
        h = lax.dot_general(h, w1_ref[i], (((1,), (1,)), ((), ())),
                            preferred_element_type=jnp.float32)
        o_ref[i:i + 1, :] = _elu(h + b1_ref[i:i + 1, :])


def _omic(x6, w0, b0, w1, b1):
    return pl.pallas_call(
        _omic_body,
        out_shape=jax.ShapeDtypeStruct((6, 256), jnp.float32),
    )(x6, w0, b0, w1, b1)


def _fc1_body(xp_ref, w_ref, b_ref, h_ref, s_ref):
    i = pl.program_id(0)
    h = lax.dot_general(xp_ref[...], w_ref[...], (((1,), (1,)), ((), ())),
                        preferred_element_type=jnp.float32)
    h = _leaky(h + b_ref[...])
    h_ref[...] = h
    ps = jnp.sum(h, axis=0, keepdims=True)

    @pl.when(i == 0)
    def _():
        s_ref[...] = ps

    @pl.when(i > 0)
    def _():
        s_ref[...] += ps


def _fc1(x_path, w, b):
    return pl.pallas_call(
        _fc1_body,
        grid=(NBLK,),
        in_specs=[
            pl.BlockSpec((BLK, DIN), lambda i: (i, 0)),
            pl.BlockSpec((D, DIN), lambda i: (0, 0)),
            pl.BlockSpec((1, D), lambda i: (0, 0)),
        ],
        out_specs=[
            pl.BlockSpec((BLK, D), lambda i: (i, 0)),
            pl.BlockSpec((1, D), lambda i: (0, 0)),
        ],
        out_shape=[
            jax.ShapeDtypeStruct((N, D), jnp.float32),
            jax.ShapeDtypeStruct((1, D), jnp.float32),
        ],
    )(x_path, w, b)


_NBDT = jnp.float16


def _proj_body(h_ref, s_ref, wh_ref, bh_ref, wt_ref, bt_ref, eh_ref, et_ref):
    x = (h_ref[...] + s_ref[...] * (1.0 / N)) * 0.5
    eh = lax.dot_general(x, wh_ref[...], (((1,), (1,)), ((), ())),
                         preferred_element_type=jnp.float32)
    eh_ref[...] = eh + bh_ref[...]
    et = lax.dot_general(x, wt_ref[...], (((1,), (1,)), ((), ())),
                         preferred_element_type=jnp.float32)
    et_ref[...] = et + bt_ref[...]


def _proj(h, s, wh, bh, wt, bt):
    return pl.pallas_call(
        _proj_body,
        grid=(NBLK,),
        in_specs=[
            pl.BlockSpec((BLK, D), lambda i: (i, 0)),
            pl.BlockSpec((1, D), lambda i: (0, 0)),
            pl.BlockSpec((D, D), lambda i: (0, 0)),
            pl.BlockSpec((1, D), lambda i: (0, 0)),
            pl.BlockSpec((D, D), lambda i: (0, 0)),
            pl.BlockSpec((1, D), lambda i: (0, 0)),
        ],
        out_specs=[
            pl.BlockSpec((BLK, D), lambda i: (i, 0)),
            pl.BlockSpec((BLK, D), lambda i: (i, 0)),
        ],
        out_shape=[
            jax.ShapeDtypeStruct((N, D), jnp.float32),
            jax.ShapeDtypeStruct((N, D), jnp.float32),
        ],
    )(h, s, wh, bh, wt, bt)


def _topk_body(eh_ref, et_ref, prob_ref, idx_ref):
    scale = D ** -0.5
    s = lax.dot_general(eh_ref[...] * scale, et_ref[...],
                        (((1,), (1,)), ((), ())),
                        preferred_element_type=jnp.float32)
    colid = lax.broadcasted_iota(jnp.int32, (BLK, N), 1)
    vals = s
    vcols = []
    icols = []
    for _ in range(K):
        m = jnp.max(vals, axis=1, keepdims=True)
        sel = vals >= m
        idx = jnp.min(jnp.where(sel, colid, jnp.int32(2 ** 30)),
                      axis=1, keepdims=True)
        vcols.append(m)
        icols.append(idx)
        vals = jnp.where(colid == idx, NEG, vals)
    v6 = jnp.concatenate(vcols, axis=1)
    mm = vcols[0]
    e6 = jnp.exp(v6 - mm)
    p6 = e6 / jnp.sum(e6, axis=1, keepdims=True)
    z1 = jnp.zeros((BLK, 1), jnp.float32)
    prob_ref[...] = jnp.concatenate([p6, z1, z1], axis=1)
    zi = jnp.zeros((BLK, 1), jnp.int32)
    idx_ref[...] = jnp.concatenate(icols + [zi, zi], axis=1)


HALF = N // 2
HBLK = HALF // BLK


def _topk(eh, et, h):
    return pl.pallas_call(
        _topk_body,
        grid=(HBLK,),
        in_specs=[
            pl.BlockSpec((BLK, D), lambda i, h=h: (i + h * HBLK, 0)),
            pl.BlockSpec((N, D), lambda i: (0, 0)),
        ],
        out_specs=[
            pl.BlockSpec((BLK, KP), lambda i: (i, 0)),
            pl.BlockSpec((BLK, KP), lambda i: (i, 0)),
        ],
        out_shape=[
            jax.ShapeDtypeStruct((HALF, KP), jnp.float32),
            jax.ShapeDtypeStruct((HALF, KP), jnp.int32),
        ],
    )(eh, et)


_GROWS = K * HALF
_GPER = _GROWS // 32
_GNB = 4
_GCH = 96
_GNCH = _GPER // _GCH


def _sc_gather_body(et_hbm, idx_hbm, out_hbm, idx_v, bufs, sems):
    wid = lax.axis_index("s") * 2 + lax.axis_index("c")
    base = wid * _GPER
    pltpu.sync_copy(idx_hbm.at[pl.ds(base, _GPER)], idx_v)
    cps = [None] * _GNCH
    for c in range(_GNB):
        cps[c] = pltpu.async_copy(
            et_hbm.at[idx_v.at[pl.ds(c * _GCH, _GCH)]], bufs[c], sems[c])
    for c in range(_GNCH):
        b = c % _GNB
        cps[c].wait()
        pltpu.sync_copy(bufs[b], out_hbm.at[pl.ds(base + c * _GCH, _GCH)])
        nxt = c + _GNB
        if nxt < _GNCH:
            cps[nxt] = pltpu.async_copy(
                et_hbm.at[idx_v.at[pl.ds(nxt * _GCH, _GCH)]], bufs[b],
                sems[b])


_PD = D // 2


def _sc_gather(et_pack, idx_flat):
    mesh = plsc.VectorSubcoreMesh(core_axis_name="c", subcore_axis_name="s")
    fn = pl.kernel(
        _sc_gather_body,
        out_type=jax.ShapeDtypeStruct((_GROWS, _PD), jnp.int32),
        mesh=mesh,
        scratch_types=[
            pltpu.VMEM((_GPER,), jnp.int32),
            [pltpu.VMEM((_GCH, _PD), jnp.int32) for _ in range(_GNB)],
            [pltpu.SemaphoreType.DMA for _ in range(_GNB)],
        ],
    )
    return fn(et_pack, idx_flat)


def _agg_body(eh_ref, nb_ref, p_ref, w1_ref, b1_ref, w2_ref, b2_ref,
              aw0_ref, ab0_ref, aw1_ref, ab1_ref, eh2_ref, g_ref):
    eh = eh_ref[...]
    nbs = [nb_ref[:, k, :] for k in range(K)]
    kws = []
    for k in range(K):
        pk = p_ref[:, k:k + 1]
        gate = jnp.tanh((2.0 - pk) * eh + pk * nbs[k])
        kws.append(jnp.sum(nbs[k], axis=1, keepdims=True)
                   * jnp.sum(gate, axis=1, keepdims=True))
    kw = jnp.concatenate(kws, axis=1)
    m = jnp.max(kw, axis=1, keepdims=True)
    e = jnp.exp(kw - m)
    sinv = 1.0 / jnp.sum(e, axis=1, keepdims=True)
    enh = (e[:, 0:1] * sinv) * nbs[0]
    for k in range(1, K):
        enh = enh + (e[:, k:k + 1] * sinv) * nbs[k]
    se = lax.dot_general(eh + enh, w1_ref[...], (((1,), (1,)), ((), ())),
                         preferred_element_type=jnp.float32)
    se = _leaky(se + b1_ref[...])
    be = lax.dot_general(eh * enh, w2_ref[...], (((1,), (1,)), ((), ())),
                         preferred_element_type=jnp.float32)
    be = _leaky(be + b2_ref[...])
    eh2 = se + be
    eh2_ref[...] = eh2
    gh = lax.dot_general(eh2, aw0_ref[...], (((1,), (1,)), ((), ())),
                         preferred_element_type=jnp.float32)
    gh = _leaky(gh + ab0_ref[...])
    g = lax.dot_general(gh, aw1_ref[...], (((1,), (1,)), ((), ())),
                        preferred_element_type=jnp.float32)
    g_ref[...] = g + ab1_ref[...]


def _agg(eh, nb, prob, w1, b1, w2, b2, aw0, ab0, aw1, ab1, h):
    return pl.pallas_call(
        _agg_body,
        grid=(HBLK,),
        in_specs=[
            pl.BlockSpec((BLK, D), lambda i, h=h: (i + h * HBLK, 0)),
            pl.BlockSpec((BLK, K, D), lambda i: (i, 0, 0)),
            pl.BlockSpec((BLK, KP), lambda i: (i, 0)),
            pl.BlockSpec((D, D), lambda i: (0, 0)),
            pl.BlockSpec((1, D), lambda i: (0, 0)),
            pl.BlockSpec((D, D), lambda i: (0, 0)),
            pl.BlockSpec((1, D), lambda i: (0, 0)),
            pl.BlockSpec((256, D), lambda i: (0, 0)),
            pl.BlockSpec((1, 256), lambda i: (0, 0)),
            pl.BlockSpec((128, 256), lambda i: (0, 0)),
            pl.BlockSpec((1, 128), lambda i: (0, 0)),
        ],
        out_specs=[
            pl.BlockSpec((BLK, D), lambda i: (i, 0)),
            pl.BlockSpec((BLK, 128), lambda i: (i, 0)),
        ],
        out_shape=[
            jax.ShapeDtypeStruct((HALF, D), jnp.float32),
            jax.ShapeDtypeStruct((HALF, 128), jnp.float32),
        ],
    )(eh, nb, prob, w1, b1, w2, b2, aw0, ab0, aw1, ab1)


def _read_body(eh2_ref, g_ref, out_ref):
    g = g_ref[:, 0:1]
    m = jnp.max(g)
    e = jnp.exp(g - m)
    w = e / jnp.sum(e)
    out_ref[...] = jnp.sum(w * eh2_ref[...], axis=0, keepdims=True)


def _read(eh2, g):
    return pl.pallas_call(
        _read_body,
        out_shape=jax.ShapeDtypeStruct((1, D), jnp.float32),
    )(eh2, g)


def kernel(x_omic1, x_omic2, x_omic3, x_omic4, x_omic5, x_omic6, x_path,
           sig0_w0, sig0_b0, sig0_w1, sig0_b1,
           sig1_w0, sig1_b0, sig1_w1, sig1_b1,
           sig2_w0, sig2_b0, sig2_w1, sig2_b1,
           sig3_w0, sig3_b0, sig3_w1, sig3_b1,
           sig4_w0, sig4_b0, sig4_w1, sig4_b1,
           sig5_w0, sig5_b0, sig5_w1, sig5_b1,
           fc1_w, fc1_b, wh_w, wh_b, wt_w, wt_b,
           lin1_w, lin1_b, lin2_w, lin2_b,
           att_w0, att_b0, att_w1, att_b1):
    xs = [x_omic1, x_omic2, x_omic3, x_omic4, x_omic5, x_omic6]
    w0s = [sig0_w0, sig1_w0, sig2_w0, sig3_w0, sig4_w0, sig5_w0]
    b0s = [sig0_b0, sig1_b0, sig2_b0, sig3_b0, sig4_b0, sig5_b0]
    w1s = [sig0_w1, sig1_w1, sig2_w1, sig3_w1, sig4_w1, sig5_w1]
    b1s = [sig0_b1, sig1_b1, sig2_b1, sig3_b1, sig4_b1, sig5_b1]
    x6 = jnp.stack([jnp.pad(x, (0, OMIC_PAD - x.shape[0])) for x in xs])
    w0 = jnp.stack([jnp.pad(w, ((0, 0), (0, OMIC_PAD - w.shape[1])))
                    for w in w0s])
    b0 = jnp.stack(b0s)
    w1 = jnp.stack(w1s)
    b1 = jnp.stack(b1s)
    e_omic = _omic(x6, w0, b0, w1, b1)[:, None, :]

    h, hsum = _fc1(x_path, fc1_w, fc1_b[None, :])
    eh, et = _proj(h, hsum, wh_w, wh_b[None, :], wt_w, wt_b[None, :])
    et16 = et.astype(_NBDT)
    aw1p = jnp.pad(att_w1, ((0, 127), (0, 0)))
    ab1p = jnp.broadcast_to(att_b1[None, :], (1, 128))
    et_pack = lax.bitcast_convert_type(et16.reshape(N, _PD, 2), jnp.int32)
    eh2s, gs = [], []
    for hh in range(2):
        prob, idx = _topk(eh, et, hh)
        idx_flat = idx[:, :K].reshape(-1)
        nb_pack = _sc_gather(et_pack, idx_flat)
        nb = (lax.bitcast_convert_type(nb_pack, _NBDT)
              .reshape(HALF, K, D).astype(jnp.float32))
        eh2_h, g_h = _agg(eh, nb, prob,
                          lin1_w, lin1_b[None, :], lin2_w, lin2_b[None, :],
                          att_w0, att_b0[None, :], aw1p, ab1p, hh)
        eh2s.append(eh2_h)
        gs.append(g_h)
    eh2 = jnp.concatenate(eh2s, axis=0)
    g = jnp.concatenate(gs, axis=0)
    e_g = _read(eh2, g)
    return (e_omic, eh2[None], e_g)


# speedup vs baseline: 4.1859x; 4.1859x over previous
import functools

import jax
import jax.numpy as jnp
from jax import lax
from jax.experimental import pallas as pl
from jax.experimental.pallas import tpu as pltpu
from jax.experimental.pallas import tpu_sc as plsc

N = 4096
DIN = 384
D = 512
K = 6
KP = 8
BLK = 256
NBLK = N // BLK
OMIC_PAD = 1536
NEG = -1e30


def _leaky(x):
    return jnp.where(x > 0, x, 0.01 * x)


def _elu(x):
    return jnp.where(x > 0, x, jnp.exp(x) - 1.0)


def _omic_body(x_ref, w0_ref, b0_ref, w1_ref, b1_ref, o_ref):
    for i in range(6):
        h = lax.dot_general(x_ref[i:i + 1, :], w0_ref[i],
                            (((1,), (1,)), ((), ())),
                            preferred_element_type=jnp.float32)
        h = _elu(h + b0_ref[i:i + 1, :])
        h = lax.dot_general(h, w1_ref[i], (((1,), (1,)), ((), ())),
                            preferred_element_type=jnp.float32)
        o_ref[i:i + 1, :] = _elu(h + b1_ref[i:i + 1, :])


def _omic(x6, w0, b0, w1, b1):
    return pl.pallas_call(
        _omic_body,
        out_shape=jax.ShapeDtypeStruct((6, 256), jnp.float32),
    )(x6, w0, b0, w1, b1)


def _fc1_body(xp_ref, w_ref, b_ref, h_ref, s_ref):
    i = pl.program_id(0)
    h = lax.dot_general(xp_ref[...], w_ref[...], (((1,), (1,)), ((), ())),
                        preferred_element_type=jnp.float32)
    h = _leaky(h + b_ref[...])
    h_ref[...] = h
    ps = jnp.sum(h, axis=0, keepdims=True)

    @pl.when(i == 0)
    def _():
        s_ref[...] = ps

    @pl.when(i > 0)
    def _():
        s_ref[...] += ps


def _fc1(x_path, w, b):
    return pl.pallas_call(
        _fc1_body,
        grid=(NBLK,),
        in_specs=[
            pl.BlockSpec((BLK, DIN), lambda i: (i, 0)),
            pl.BlockSpec((D, DIN), lambda i: (0, 0)),
            pl.BlockSpec((1, D), lambda i: (0, 0)),
        ],
        out_specs=[
            pl.BlockSpec((BLK, D), lambda i: (i, 0)),
            pl.BlockSpec((1, D), lambda i: (0, 0)),
        ],
        out_shape=[
            jax.ShapeDtypeStruct((N, D), jnp.float32),
            jax.ShapeDtypeStruct((1, D), jnp.float32),
        ],
    )(x_path, w, b)


_PD = D // 2


def _f16_enc(x):
    y = lax.bitcast_convert_type(x * jnp.float32(2.0 ** -112), jnp.int32)
    y = y + 0xFFF + ((y >> 13) & 1)
    return ((y >> 16) & 0x8000) | ((y >> 13) & 0x7FFF)


def _f16_dec(h):
    z = ((h & 0x8000) << 16) | ((h & 0x7FFF) << 13)
    return lax.bitcast_convert_type(z, jnp.float32) * jnp.float32(2.0 ** 112)


def _proj_body(h_ref, s_ref, wh_ref, bh_ref, wt_ref, bt_ref, eh_ref, et_ref,
               pk_ref):
    x = (h_ref[...] + s_ref[...] * (1.0 / N)) * 0.5
    eh = lax.dot_general(x, wh_ref[...], (((1,), (1,)), ((), ())),
                         preferred_element_type=jnp.float32)
    eh_ref[...] = eh + bh_ref[...]
    et = lax.dot_general(x, wt_ref[...], (((1,), (1,)), ((), ())),
                         preferred_element_type=jnp.float32)
    et = et + bt_ref[...]
    et_ref[...] = et
    lo = _f16_enc(et[:, :_PD])
    hi = _f16_enc(et[:, _PD:])
    pk_ref[...] = lo | (hi << 16)


def _proj(h, s, wh, bh, wt, bt):
    return pl.pallas_call(
        _proj_body,
        grid=(NBLK,),
        in_specs=[
            pl.BlockSpec((BLK, D), lambda i: (i, 0)),
            pl.BlockSpec((1, D), lambda i: (0, 0)),
            pl.BlockSpec((D, D), lambda i: (0, 0)),
            pl.BlockSpec((1, D), lambda i: (0, 0)),
            pl.BlockSpec((D, D), lambda i: (0, 0)),
            pl.BlockSpec((1, D), lambda i: (0, 0)),
        ],
        out_specs=[
            pl.BlockSpec((BLK, D), lambda i: (i, 0)),
            pl.BlockSpec((BLK, D), lambda i: (i, 0)),
            pl.BlockSpec((BLK, _PD), lambda i: (i, 0)),
        ],
        out_shape=[
            jax.ShapeDtypeStruct((N, D), jnp.float32),
            jax.ShapeDtypeStruct((N, D), jnp.float32),
            jax.ShapeDtypeStruct((N, _PD), jnp.int32),
        ],
    )(h, s, wh, bh, wt, bt)


def _topk_body(eh_ref, et_ref, prob_ref, idx_ref):
    scale = D ** -0.5
    s = lax.dot_general(eh_ref[...] * scale, et_ref[...],
                        (((1,), (1,)), ((), ())),
                        preferred_element_type=jnp.float32)
    colid = lax.broadcasted_iota(jnp.int32, (BLK, N), 1)
    vals = s
    vcols = []
    icols = []
    for _ in range(K):
        m = jnp.max(vals, axis=1, keepdims=True)
        sel = vals >= m
        idx = jnp.min(jnp.where(sel, colid, jnp.int32(2 ** 30)),
                      axis=1, keepdims=True)
        vcols.append(m)
        icols.append(idx)
        vals = jnp.where(colid == idx, NEG, vals)
    v6 = jnp.concatenate(vcols, axis=1)
    mm = vcols[0]
    e6 = jnp.exp(v6 - mm)
    p6 = e6 / jnp.sum(e6, axis=1, keepdims=True)
    z1 = jnp.zeros((BLK, 1), jnp.float32)
    prob_ref[...] = jnp.concatenate([p6, z1, z1], axis=1)
    zi = jnp.zeros((BLK, 1), jnp.int32)
    idx_ref[...] = jnp.concatenate(icols + [zi, zi], axis=1)


HALF = N // 2
HBLK = HALF // BLK


def _topk(eh, et, h):
    return pl.pallas_call(
        _topk_body,
        grid=(HBLK,),
        in_specs=[
            pl.BlockSpec((BLK, D), lambda i, h=h: (i + h * HBLK, 0)),
            pl.BlockSpec((N, D), lambda i: (0, 0)),
        ],
        out_specs=[
            pl.BlockSpec((BLK, KP), lambda i: (i, 0)),
            pl.BlockSpec((BLK, KP), lambda i: (i, 0)),
        ],
        out_shape=[
            jax.ShapeDtypeStruct((HALF, KP), jnp.float32),
            jax.ShapeDtypeStruct((HALF, KP), jnp.int32),
        ],
    )(eh, et)


_GROWS = K * HALF
_GPER = _GROWS // 32
_GNB = 4
_GCH = 96
_GNCH = _GPER // _GCH


def _sc_gather_body(et_hbm, idx_hbm, out_hbm, idx_v, bufs, sems):
    wid = lax.axis_index("s") * 2 + lax.axis_index("c")
    base = wid * _GPER
    pltpu.sync_copy(idx_hbm.at[pl.ds(base, _GPER)], idx_v)
    cps = [None] * _GNCH
    for c in range(_GNB):
        cps[c] = pltpu.async_copy(
            et_hbm.at[idx_v.at[pl.ds(c * _GCH, _GCH)]], bufs[c], sems[c])
    for c in range(_GNCH):
        b = c % _GNB
        cps[c].wait()
        pltpu.sync_copy(bufs[b], out_hbm.at[pl.ds(base + c * _GCH, _GCH)])
        nxt = c + _GNB
        if nxt < _GNCH:
            cps[nxt] = pltpu.async_copy(
                et_hbm.at[idx_v.at[pl.ds(nxt * _GCH, _GCH)]], bufs[b],
                sems[b])


def _sc_gather(et_pack, idx_flat):
    mesh = plsc.VectorSubcoreMesh(core_axis_name="c", subcore_axis_name="s")
    fn = pl.kernel(
        _sc_gather_body,
        out_type=jax.ShapeDtypeStruct((_GROWS, _PD), jnp.int32),
        mesh=mesh,
        scratch_types=[
            pltpu.VMEM((_GPER,), jnp.int32),
            [pltpu.VMEM((_GCH, _PD), jnp.int32) for _ in range(_GNB)],
            [pltpu.SemaphoreType.DMA for _ in range(_GNB)],
        ],
    )
    return fn(et_pack, idx_flat)


def _agg_body(eh_ref, nb_ref, p_ref, w1_ref, b1_ref, w2_ref, b2_ref,
              aw0_ref, ab0_ref, aw1_ref, ab1_ref, eh2_ref, g_ref):
    eh = eh_ref[...]
    nbs = []
    for k in range(K):
        p = nb_ref[:, k, :]
        a = _f16_dec(p & 0xFFFF)
        b = _f16_dec((p >> 16) & 0xFFFF)
        nbs.append(jnp.concatenate([a, b], axis=1))
    kws = []
    for k in range(K):
        pk = p_ref[:, k:k + 1]
        gate = jnp.tanh((2.0 - pk) * eh + pk * nbs[k])
        kws.append(jnp.sum(nbs[k], axis=1, keepdims=True)
                   * jnp.sum(gate, axis=1, keepdims=True))
    kw = jnp.concatenate(kws, axis=1)
    m = jnp.max(kw, axis=1, keepdims=True)
    e = jnp.exp(kw - m)
    sinv = 1.0 / jnp.sum(e, axis=1, keepdims=True)
    enh = (e[:, 0:1] * sinv) * nbs[0]
    for k in range(1, K):
        enh = enh + (e[:, k:k + 1] * sinv) * nbs[k]
    se = lax.dot_general(eh + enh, w1_ref[...], (((1,), (1,)), ((), ())),
                         preferred_element_type=jnp.float32)
    se = _leaky(se + b1_ref[...])
    be = lax.dot_general(eh * enh, w2_ref[...], (((1,), (1,)), ((), ())),
                         preferred_element_type=jnp.float32)
    be = _leaky(be + b2_ref[...])
    eh2 = se + be
    eh2_ref[...] = eh2
    gh = lax.dot_general(eh2, aw0_ref[...], (((1,), (1,)), ((), ())),
                         preferred_element_type=jnp.float32)
    gh = _leaky(gh + ab0_ref[...])
    g = lax.dot_general(gh, aw1_ref[...], (((1,), (1,)), ((), ())),
                        preferred_element_type=jnp.float32)
    g_ref[...] = g + ab1_ref[...]


def _agg(eh, nb, prob, w1, b1, w2, b2, aw0, ab0, aw1, ab1, h):
    return pl.pallas_call(
        _agg_body,
        grid=(HBLK,),
        in_specs=[
            pl.BlockSpec((BLK, D), lambda i, h=h: (i + h * HBLK, 0)),
            pl.BlockSpec((BLK, K, _PD), lambda i: (i, 0, 0)),
            pl.BlockSpec((BLK, KP), lambda i: (i, 0)),
            pl.BlockSpec((D, D), lambda i: (0, 0)),
            pl.BlockSpec((1, D), lambda i: (0, 0)),
            pl.BlockSpec((D, D), lambda i: (0, 0)),
            pl.BlockSpec((1, D), lambda i: (0, 0)),
            pl.BlockSpec((256, D), lambda i: (0, 0)),
            pl.BlockSpec((1, 256), lambda i: (0, 0)),
            pl.BlockSpec((128, 256), lambda i: (0, 0)),
            pl.BlockSpec((1, 128), lambda i: (0, 0)),
        ],
        out_specs=[
            pl.BlockSpec((BLK, D), lambda i: (i, 0)),
            pl.BlockSpec((BLK, 128), lambda i: (i, 0)),
        ],
        out_shape=[
            jax.ShapeDtypeStruct((HALF, D), jnp.float32),
            jax.ShapeDtypeStruct((HALF, 128), jnp.float32),
        ],
    )(eh, nb, prob, w1, b1, w2, b2, aw0, ab0, aw1, ab1)


def _read_body(eh2_ref, g_ref, out_ref):
    g = g_ref[:, 0:1]
    m = jnp.max(g)
    e = jnp.exp(g - m)
    w = e / jnp.sum(e)
    out_ref[...] = jnp.sum(w * eh2_ref[...], axis=0, keepdims=True)


def _read(eh2, g):
    return pl.pallas_call(
        _read_body,
        out_shape=jax.ShapeDtypeStruct((1, D), jnp.float32),
    )(eh2, g)


def kernel(x_omic1, x_omic2, x_omic3, x_omic4, x_omic5, x_omic6, x_path,
           sig0_w0, sig0_b0, sig0_w1, sig0_b1,
           sig1_w0, sig1_b0, sig1_w1, sig1_b1,
           sig2_w0, sig2_b0, sig2_w1, sig2_b1,
           sig3_w0, sig3_b0, sig3_w1, sig3_b1,
           sig4_w0, sig4_b0, sig4_w1, sig4_b1,
           sig5_w0, sig5_b0, sig5_w1, sig5_b1,
           fc1_w, fc1_b, wh_w, wh_b, wt_w, wt_b,
           lin1_w, lin1_b, lin2_w, lin2_b,
           att_w0, att_b0, att_w1, att_b1):
    xs = [x_omic1, x_omic2, x_omic3, x_omic4, x_omic5, x_omic6]
    w0s = [sig0_w0, sig1_w0, sig2_w0, sig3_w0, sig4_w0, sig5_w0]
    b0s = [sig0_b0, sig1_b0, sig2_b0, sig3_b0, sig4_b0, sig5_b0]
    w1s = [sig0_w1, sig1_w1, sig2_w1, sig3_w1, sig4_w1, sig5_w1]
    b1s = [sig0_b1, sig1_b1, sig2_b1, sig3_b1, sig4_b1, sig5_b1]
    x6 = jnp.stack([jnp.pad(x, (0, OMIC_PAD - x.shape[0])) for x in xs])
    w0 = jnp.stack([jnp.pad(w, ((0, 0), (0, OMIC_PAD - w.shape[1])))
                    for w in w0s])
    b0 = jnp.stack(b0s)
    w1 = jnp.stack(w1s)
    b1 = jnp.stack(b1s)
    e_omic = _omic(x6, w0, b0, w1, b1)[:, None, :]

    h, hsum = _fc1(x_path, fc1_w, fc1_b[None, :])
    eh, et, et_pack = _proj(h, hsum, wh_w, wh_b[None, :], wt_w, wt_b[None, :])
    aw1p = jnp.pad(att_w1, ((0, 127), (0, 0)))
    ab1p = jnp.broadcast_to(att_b1[None, :], (1, 128))
    eh2s, gs = [], []
    for hh in range(2):
        prob, idx = _topk(eh, et, hh)
        idx_flat = idx[:, :K].reshape(-1)
        nb = _sc_gather(et_pack, idx_flat).reshape(HALF, K, _PD)
        eh2_h, g_h = _agg(eh, nb, prob,
                          lin1_w, lin1_b[None, :], lin2_w, lin2_b[None, :],
                          att_w0, att_b0[None, :], aw1p, ab1p, hh)
        eh2s.append(eh2_h)
        gs.append(g_h)
    eh2 = jnp.concatenate(eh2s, axis=0)
    g = jnp.concatenate(gs, axis=0)
    e_g = _read(eh2, g)
    return (e_omic, eh2[None], e_g)


# speedup vs baseline: 4.9685x; 1.1870x over previous
import functools

import jax
import jax.numpy as jnp
from jax import lax
from jax.experimental import pallas as pl
from jax.experimental.pallas import tpu as pltpu
from jax.experimental.pallas import tpu_sc as plsc

N = 4096
DIN = 384
D = 512
K = 6
KP = 8
BLK = 256
NBLK = N // BLK
OMIC_PAD = 1536
NEG = -1e30


def _leaky(x):
    return jnp.where(x > 0, x, 0.01 * x)


def _elu(x):
    return jnp.where(x > 0, x, jnp.exp(x) - 1.0)


def _omic_body(x_ref, w0_ref, b0_ref, w1_ref, b1_ref, o_ref):
    for i in range(6):
        h = lax.dot_general(x_ref[i:i + 1, :], w0_ref[i],
                            (((1,), (1,)), ((), ())),
                            preferred_element_type=jnp.float32)
        h = _elu(h + b0_ref[i:i + 1, :])
        h = lax.dot_general(h, w1_ref[i], (((1,), (1,)), ((), ())),
                            preferred_element_type=jnp.float32)
        o_ref[i:i + 1, :] = _elu(h + b1_ref[i:i + 1, :])


def _omic(x6, w0, b0, w1, b1):
    return pl.pallas_call(
        _omic_body,
        out_shape=jax.ShapeDtypeStruct((6, 256), jnp.float32),
    )(x6, w0, b0, w1, b1)


def _fc1_body(xp_ref, w_ref, b_ref, h_ref, s_ref):
    i = pl.program_id(0)
    h = lax.dot_general(xp_ref[...], w_ref[...], (((1,), (1,)), ((), ())),
                        preferred_element_type=jnp.float32)
    h = _leaky(h + b_ref[...])
    h_ref[...] = h
    ps = jnp.sum(h, axis=0, keepdims=True)

    @pl.when(i == 0)
    def _():
        s_ref[...] = ps

    @pl.when(i > 0)
    def _():
        s_ref[...] += ps


def _fc1(x_path, w, b):
    return pl.pallas_call(
        _fc1_body,
        grid=(NBLK,),
        in_specs=[
            pl.BlockSpec((BLK, DIN), lambda i: (i, 0)),
            pl.BlockSpec((D, DIN), lambda i: (0, 0)),
            pl.BlockSpec((1, D), lambda i: (0, 0)),
        ],
        out_specs=[
            pl.BlockSpec((BLK, D), lambda i: (i, 0)),
            pl.BlockSpec((1, D), lambda i: (0, 0)),
        ],
        out_shape=[
            jax.ShapeDtypeStruct((N, D), jnp.float32),
            jax.ShapeDtypeStruct((1, D), jnp.float32),
        ],
    )(x_path, w, b)


_PD = D // 2


def _f16_enc(x):
    y = lax.bitcast_convert_type(x * jnp.float32(2.0 ** -112), jnp.int32)
    y = y + 0xFFF + ((y >> 13) & 1)
    return ((y >> 16) & 0x8000) | ((y >> 13) & 0x7FFF)


def _f16_dec(h):
    z = ((h & 0x8000) << 16) | ((h & 0x7FFF) << 13)
    return lax.bitcast_convert_type(z, jnp.float32) * jnp.float32(2.0 ** 112)


def _proj_body(h_ref, s_ref, wh_ref, bh_ref, wt_ref, bt_ref, eh_ref, et_ref,
               pk_ref):
    x = (h_ref[...] + s_ref[...] * (1.0 / N)) * 0.5
    eh = lax.dot_general(x, wh_ref[...], (((1,), (1,)), ((), ())),
                         preferred_element_type=jnp.float32)
    eh_ref[...] = eh + bh_ref[...]
    et = lax.dot_general(x, wt_ref[...], (((1,), (1,)), ((), ())),
                         preferred_element_type=jnp.float32)
    et = et + bt_ref[...]
    et_ref[...] = et
    lo = _f16_enc(et[:, :_PD])
    hi = _f16_enc(et[:, _PD:])
    pk_ref[...] = lo | (hi << 16)


def _proj(h, s, wh, bh, wt, bt):
    return pl.pallas_call(
        _proj_body,
        grid=(NBLK,),
        in_specs=[
            pl.BlockSpec((BLK, D), lambda i: (i, 0)),
            pl.BlockSpec((1, D), lambda i: (0, 0)),
            pl.BlockSpec((D, D), lambda i: (0, 0)),
            pl.BlockSpec((1, D), lambda i: (0, 0)),
            pl.BlockSpec((D, D), lambda i: (0, 0)),
            pl.BlockSpec((1, D), lambda i: (0, 0)),
        ],
        out_specs=[
            pl.BlockSpec((BLK, D), lambda i: (i, 0)),
            pl.BlockSpec((BLK, D), lambda i: (i, 0)),
            pl.BlockSpec((BLK, _PD), lambda i: (i, 0)),
        ],
        out_shape=[
            jax.ShapeDtypeStruct((N, D), jnp.float32),
            jax.ShapeDtypeStruct((N, D), jnp.float32),
            jax.ShapeDtypeStruct((N, _PD), jnp.int32),
        ],
    )(h, s, wh, bh, wt, bt)


def _topk_body(eh_ref, et_ref, prob_ref, idx_ref):
    scale = D ** -0.5
    s = lax.dot_general(eh_ref[...] * scale, et_ref[...],
                        (((1,), (1,)), ((), ())),
                        preferred_element_type=jnp.float32)
    colid = lax.broadcasted_iota(jnp.int32, (BLK, N), 1)
    vals = s
    vcols = []
    icols = []
    for _ in range(K):
        m = jnp.max(vals, axis=1, keepdims=True)
        sel = vals >= m
        idx = jnp.min(jnp.where(sel, colid, jnp.int32(2 ** 30)),
                      axis=1, keepdims=True)
        vcols.append(m)
        icols.append(idx)
        vals = jnp.where(colid == idx, NEG, vals)
    v6 = jnp.concatenate(vcols, axis=1)
    mm = vcols[0]
    e6 = jnp.exp(v6 - mm)
    p6 = e6 / jnp.sum(e6, axis=1, keepdims=True)
    z1 = jnp.zeros((BLK, 1), jnp.float32)
    prob_ref[...] = jnp.concatenate([p6, z1, z1], axis=1)
    zi = jnp.zeros((BLK, 1), jnp.int32)
    idx_ref[...] = jnp.concatenate(icols + [zi, zi], axis=1)


HALF = N // 2
HBLK = HALF // BLK


def _topk(eh, et, h):
    return pl.pallas_call(
        _topk_body,
        grid=(HBLK,),
        in_specs=[
            pl.BlockSpec((BLK, D), lambda i, h=h: (i + h * HBLK, 0)),
            pl.BlockSpec((N, D), lambda i: (0, 0)),
        ],
        out_specs=[
            pl.BlockSpec((BLK, KP), lambda i: (i, 0)),
            pl.BlockSpec((BLK, KP), lambda i: (i, 0)),
        ],
        out_shape=[
            jax.ShapeDtypeStruct((HALF, KP), jnp.float32),
            jax.ShapeDtypeStruct((HALF, KP), jnp.int32),
        ],
    )(eh, et)


_GROWS = K * HALF
_GPER = _GROWS // 32
_GNB = 4
_GCH = 96
_GNCH = _GPER // _GCH


def _sc_gather_body(et_hbm, idx_hbm, out_hbm, idx_v, bufs, sems):
    wid = lax.axis_index("s") * 2 + lax.axis_index("c")
    base = wid * _GPER
    pltpu.sync_copy(idx_hbm.at[pl.ds(base, _GPER)], idx_v)
    cps = [None] * _GNCH
    for c in range(_GNB):
        cps[c] = pltpu.async_copy(
            et_hbm.at[idx_v.at[pl.ds(c * _GCH, _GCH)]], bufs[c], sems[c])
    for c in range(_GNCH):
        b = c % _GNB
        cps[c].wait()
        pltpu.sync_copy(bufs[b], out_hbm.at[pl.ds(base + c * _GCH, _GCH)])
        nxt = c + _GNB
        if nxt < _GNCH:
            cps[nxt] = pltpu.async_copy(
                et_hbm.at[idx_v.at[pl.ds(nxt * _GCH, _GCH)]], bufs[b],
                sems[b])


def _sc_gather(et_pack, idx_flat):
    mesh = plsc.VectorSubcoreMesh(core_axis_name="c", subcore_axis_name="s")
    fn = pl.kernel(
        _sc_gather_body,
        out_type=jax.ShapeDtypeStruct((_GROWS, _PD), jnp.int32),
        mesh=mesh,
        scratch_types=[
            pltpu.VMEM((_GPER,), jnp.int32),
            [pltpu.VMEM((_GCH, _PD), jnp.int32) for _ in range(_GNB)],
            [pltpu.SemaphoreType.DMA for _ in range(_GNB)],
        ],
    )
    return fn(et_pack, idx_flat)


def _agg_body(eh_ref, nb_ref, p_ref, w1_ref, b1_ref, w2_ref, b2_ref,
              aw0_ref, ab0_ref, aw1_ref, ab1_ref, eh2_ref, g_ref):
    eh = eh_ref[...]
    nbs = []
    for k in range(K):
        p = nb_ref[k]
        a = _f16_dec(p & 0xFFFF)
        b = _f16_dec((p >> 16) & 0xFFFF)
        nbs.append(jnp.concatenate([a, b], axis=1))
    kws = []
    for k in range(K):
        pk = p_ref[:, k:k + 1]
        gate = jnp.tanh((2.0 - pk) * eh + pk * nbs[k])
        kws.append(jnp.sum(nbs[k], axis=1, keepdims=True)
                   * jnp.sum(gate, axis=1, keepdims=True))
    kw = jnp.concatenate(kws, axis=1)
    m = jnp.max(kw, axis=1, keepdims=True)
    e = jnp.exp(kw - m)
    sinv = 1.0 / jnp.sum(e, axis=1, keepdims=True)
    enh = (e[:, 0:1] * sinv) * nbs[0]
    for k in range(1, K):
        enh = enh + (e[:, k:k + 1] * sinv) * nbs[k]
    se = lax.dot_general(eh + enh, w1_ref[...], (((1,), (1,)), ((), ())),
                         preferred_element_type=jnp.float32)
    se = _leaky(se + b1_ref[...])
    be = lax.dot_general(eh * enh, w2_ref[...], (((1,), (1,)), ((), ())),
                         preferred_element_type=jnp.float32)
    be = _leaky(be + b2_ref[...])
    eh2 = se + be
    eh2_ref[...] = eh2
    gh = lax.dot_general(eh2, aw0_ref[...], (((1,), (1,)), ((), ())),
                         preferred_element_type=jnp.float32)
    gh = _leaky(gh + ab0_ref[...])
    g = lax.dot_general(gh, aw1_ref[...], (((1,), (1,)), ((), ())),
                        preferred_element_type=jnp.float32)
    g_ref[...] = g + ab1_ref[...]


def _agg(eh, nb, prob, w1, b1, w2, b2, aw0, ab0, aw1, ab1, h):
    return pl.pallas_call(
        _agg_body,
        grid=(HBLK,),
        in_specs=[
            pl.BlockSpec((BLK, D), lambda i, h=h: (i + h * HBLK, 0)),
            pl.BlockSpec((K, BLK, _PD), lambda i: (0, i, 0)),
            pl.BlockSpec((BLK, KP), lambda i: (i, 0)),
            pl.BlockSpec((D, D), lambda i: (0, 0)),
            pl.BlockSpec((1, D), lambda i: (0, 0)),
            pl.BlockSpec((D, D), lambda i: (0, 0)),
            pl.BlockSpec((1, D), lambda i: (0, 0)),
            pl.BlockSpec((256, D), lambda i: (0, 0)),
            pl.BlockSpec((1, 256), lambda i: (0, 0)),
            pl.BlockSpec((128, 256), lambda i: (0, 0)),
            pl.BlockSpec((1, 128), lambda i: (0, 0)),
        ],
        out_specs=[
            pl.BlockSpec((BLK, D), lambda i: (i, 0)),
            pl.BlockSpec((BLK, 128), lambda i: (i, 0)),
        ],
        out_shape=[
            jax.ShapeDtypeStruct((HALF, D), jnp.float32),
            jax.ShapeDtypeStruct((HALF, 128), jnp.float32),
        ],
    )(eh, nb, prob, w1, b1, w2, b2, aw0, ab0, aw1, ab1)


def _read_body(eh2_ref, g_ref, out_ref):
    g = g_ref[:, 0:1]
    m = jnp.max(g)
    e = jnp.exp(g - m)
    w = e / jnp.sum(e)
    out_ref[...] = jnp.sum(w * eh2_ref[...], axis=0, keepdims=True)


def _read(eh2, g):
    return pl.pallas_call(
        _read_body,
        out_shape=jax.ShapeDtypeStruct((1, D), jnp.float32),
    )(eh2, g)


def kernel(x_omic1, x_omic2, x_omic3, x_omic4, x_omic5, x_omic6, x_path,
           sig0_w0, sig0_b0, sig0_w1, sig0_b1,
           sig1_w0, sig1_b0, sig1_w1, sig1_b1,
           sig2_w0, sig2_b0, sig2_w1, sig2_b1,
           sig3_w0, sig3_b0, sig3_w1, sig3_b1,
           sig4_w0, sig4_b0, sig4_w1, sig4_b1,
           sig5_w0, sig5_b0, sig5_w1, sig5_b1,
           fc1_w, fc1_b, wh_w, wh_b, wt_w, wt_b,
           lin1_w, lin1_b, lin2_w, lin2_b,
           att_w0, att_b0, att_w1, att_b1):
    xs = [x_omic1, x_omic2, x_omic3, x_omic4, x_omic5, x_omic6]
    w0s = [sig0_w0, sig1_w0, sig2_w0, sig3_w0, sig4_w0, sig5_w0]
    b0s = [sig0_b0, sig1_b0, sig2_b0, sig3_b0, sig4_b0, sig5_b0]
    w1s = [sig0_w1, sig1_w1, sig2_w1, sig3_w1, sig4_w1, sig5_w1]
    b1s = [sig0_b1, sig1_b1, sig2_b1, sig3_b1, sig4_b1, sig5_b1]
    x6 = jnp.stack([jnp.pad(x, (0, OMIC_PAD - x.shape[0])) for x in xs])
    w0 = jnp.stack([jnp.pad(w, ((0, 0), (0, OMIC_PAD - w.shape[1])))
                    for w in w0s])
    b0 = jnp.stack(b0s)
    w1 = jnp.stack(w1s)
    b1 = jnp.stack(b1s)
    e_omic = _omic(x6, w0, b0, w1, b1)[:, None, :]

    h, hsum = _fc1(x_path, fc1_w, fc1_b[None, :])
    eh, et, et_pack = _proj(h, hsum, wh_w, wh_b[None, :], wt_w, wt_b[None, :])
    aw1p = jnp.pad(att_w1, ((0, 127), (0, 0)))
    ab1p = jnp.broadcast_to(att_b1[None, :], (1, 128))
    eh2s, gs = [], []
    for hh in range(2):
        prob, idx = _topk(eh, et, hh)
        idx_flat = jnp.transpose(idx)[:K].reshape(-1)
        nb = _sc_gather(et_pack, idx_flat).reshape(K, HALF, _PD)
        eh2_h, g_h = _agg(eh, nb, prob,
                          lin1_w, lin1_b[None, :], lin2_w, lin2_b[None, :],
                          att_w0, att_b0[None, :], aw1p, ab1p, hh)
        eh2s.append(eh2_h)
        gs.append(g_h)
    eh2 = jnp.concatenate(eh2s, axis=0)
    g = jnp.concatenate(gs, axis=0)
    e_g = _read(eh2, g)
    return (e_omic, eh2[None], e_g)


# speedup vs baseline: 5.4221x; 1.0913x over previous
import functools

import jax
import jax.numpy as jnp
from jax import lax
from jax.experimental import pallas as pl
from jax.experimental.pallas import tpu as pltpu
from jax.experimental.pallas import tpu_sc as plsc

N = 4096
DIN = 384
D = 512
K = 6
KP = 8
BLK = 256
NBLK = N // BLK
OMIC_PAD = 1536
NEG = -1e30


def _leaky(x):
    return jnp.where(x > 0, x, 0.01 * x)


def _elu(x):
    return jnp.where(x > 0, x, jnp.exp(x) - 1.0)


def _omic_body(*refs):
    o_ref = refs[-1]
    for i in range(6):
        x, w0, b0, w1, b1 = refs[5 * i:5 * i + 5]
        h = lax.dot_general(x[...], w0[...], (((1,), (1,)), ((), ())),
                            preferred_element_type=jnp.float32)
        h = _elu(h + b0[...])
        h = lax.dot_general(h, w1[...], (((1,), (1,)), ((), ())),
                            preferred_element_type=jnp.float32)
        o_ref[i:i + 1, :] = _elu(h + b1[...])


def _omic(args):
    return pl.pallas_call(
        _omic_body,
        out_shape=jax.ShapeDtypeStruct((6, 256), jnp.float32),
    )(*args)


def _fc1_body(xp_ref, w_ref, b_ref, h_ref, s_ref):
    i = pl.program_id(0)
    h = lax.dot_general(xp_ref[...], w_ref[...], (((1,), (1,)), ((), ())),
                        preferred_element_type=jnp.float32)
    h = _leaky(h + b_ref[...])
    h_ref[...] = h
    ps = jnp.sum(h, axis=0, keepdims=True)

    @pl.when(i == 0)
    def _():
        s_ref[...] = ps

    @pl.when(i > 0)
    def _():
        s_ref[...] += ps


def _fc1(x_path, w, b):
    return pl.pallas_call(
        _fc1_body,
        grid=(NBLK,),
        in_specs=[
            pl.BlockSpec((BLK, DIN), lambda i: (i, 0)),
            pl.BlockSpec((D, DIN), lambda i: (0, 0)),
            pl.BlockSpec((1, D), lambda i: (0, 0)),
        ],
        out_specs=[
            pl.BlockSpec((BLK, D), lambda i: (i, 0)),
            pl.BlockSpec((1, D), lambda i: (0, 0)),
        ],
        out_shape=[
            jax.ShapeDtypeStruct((N, D), jnp.float32),
            jax.ShapeDtypeStruct((1, D), jnp.float32),
        ],
    )(x_path, w, b)


_PD = D // 2


def _f16_enc(x):
    y = lax.bitcast_convert_type(x * jnp.float32(2.0 ** -112), jnp.int32)
    y = y + 0xFFF + ((y >> 13) & 1)
    return ((y >> 16) & 0x8000) | ((y >> 13) & 0x7FFF)


def _f16_dec(h):
    z = ((h & 0x8000) << 16) | ((h & 0x7FFF) << 13)
    return lax.bitcast_convert_type(z, jnp.float32) * jnp.float32(2.0 ** 112)


def _proj_body(h_ref, s_ref, wh_ref, bh_ref, wt_ref, bt_ref, eh_ref, et_ref,
               pk_ref):
    x = (h_ref[...] + s_ref[...] * (1.0 / N)) * 0.5
    eh = lax.dot_general(x, wh_ref[...], (((1,), (1,)), ((), ())),
                         preferred_element_type=jnp.float32)
    eh_ref[...] = eh + bh_ref[...]
    et = lax.dot_general(x, wt_ref[...], (((1,), (1,)), ((), ())),
                         preferred_element_type=jnp.float32)
    et = et + bt_ref[...]
    et_ref[...] = et
    lo = _f16_enc(et[:, :_PD])
    hi = _f16_enc(et[:, _PD:])
    pk_ref[...] = lo | (hi << 16)


def _proj(h, s, wh, bh, wt, bt):
    return pl.pallas_call(
        _proj_body,
        grid=(NBLK,),
        in_specs=[
            pl.BlockSpec((BLK, D), lambda i: (i, 0)),
            pl.BlockSpec((1, D), lambda i: (0, 0)),
            pl.BlockSpec((D, D), lambda i: (0, 0)),
            pl.BlockSpec((1, D), lambda i: (0, 0)),
            pl.BlockSpec((D, D), lambda i: (0, 0)),
            pl.BlockSpec((1, D), lambda i: (0, 0)),
        ],
        out_specs=[
            pl.BlockSpec((BLK, D), lambda i: (i, 0)),
            pl.BlockSpec((BLK, D), lambda i: (i, 0)),
            pl.BlockSpec((BLK, _PD), lambda i: (i, 0)),
        ],
        out_shape=[
            jax.ShapeDtypeStruct((N, D), jnp.float32),
            jax.ShapeDtypeStruct((N, D), jnp.float32),
            jax.ShapeDtypeStruct((N, _PD), jnp.int32),
        ],
    )(h, s, wh, bh, wt, bt)


def _topk_body(eh_ref, et_ref, prob_ref, idx_ref):
    scale = D ** -0.5
    s = lax.dot_general(eh_ref[...] * scale, et_ref[...],
                        (((1,), (1,)), ((), ())),
                        preferred_element_type=jnp.float32)
    colid = lax.broadcasted_iota(jnp.int32, (BLK, N), 1)
    vals = s
    vcols = []
    icols = []
    for _ in range(K):
        m = jnp.max(vals, axis=1, keepdims=True)
        sel = vals >= m
        idx = jnp.min(jnp.where(sel, colid, jnp.int32(2 ** 30)),
                      axis=1, keepdims=True)
        vcols.append(m)
        icols.append(idx)
        vals = jnp.where(colid == idx, NEG, vals)
    v6 = jnp.concatenate(vcols, axis=1)
    mm = vcols[0]
    e6 = jnp.exp(v6 - mm)
    p6 = e6 / jnp.sum(e6, axis=1, keepdims=True)
    z1 = jnp.zeros((BLK, 1), jnp.float32)
    prob_ref[...] = jnp.concatenate([p6, z1, z1], axis=1)
    zi = jnp.zeros((BLK, 1), jnp.int32)
    idx_ref[...] = jnp.concatenate(icols + [zi, zi], axis=1)


NSPLIT = 4
HALF = N // NSPLIT
HBLK = HALF // BLK


def _topk(eh, et, h):
    return pl.pallas_call(
        _topk_body,
        grid=(HBLK,),
        in_specs=[
            pl.BlockSpec((BLK, D), lambda i, h=h: (i + h * HBLK, 0)),
            pl.BlockSpec((N, D), lambda i: (0, 0)),
        ],
        out_specs=[
            pl.BlockSpec((BLK, KP), lambda i: (i, 0)),
            pl.BlockSpec((BLK, KP), lambda i: (i, 0)),
        ],
        out_shape=[
            jax.ShapeDtypeStruct((HALF, KP), jnp.float32),
            jax.ShapeDtypeStruct((HALF, KP), jnp.int32),
        ],
    )(eh, et)


_GROWS = K * HALF
_GPER = _GROWS // 32
_GNB = 2
_GCH = 96
_GNCH = _GPER // _GCH


def _sc_gather_body(et_hbm, idx_hbm, out_hbm, idx_v, bufs, sems):
    wid = lax.axis_index("s") * 2 + lax.axis_index("c")
    base = wid * _GPER
    pltpu.sync_copy(idx_hbm.at[pl.ds(base, _GPER)], idx_v)
    cps = [None] * _GNCH
    for c in range(_GNB):
        cps[c] = pltpu.async_copy(
            et_hbm.at[idx_v.at[pl.ds(c * _GCH, _GCH)]], bufs[c], sems[c])
    for c in range(_GNCH):
        b = c % _GNB
        cps[c].wait()
        pltpu.sync_copy(bufs[b], out_hbm.at[pl.ds(base + c * _GCH, _GCH)])
        nxt = c + _GNB
        if nxt < _GNCH:
            cps[nxt] = pltpu.async_copy(
                et_hbm.at[idx_v.at[pl.ds(nxt * _GCH, _GCH)]], bufs[b],
                sems[b])


def _sc_gather(et_pack, idx_flat):
    mesh = plsc.VectorSubcoreMesh(core_axis_name="c", subcore_axis_name="s")
    fn = pl.kernel(
        _sc_gather_body,
        out_type=jax.ShapeDtypeStruct((_GROWS, _PD), jnp.int32),
        mesh=mesh,
        scratch_types=[
            pltpu.VMEM((_GPER,), jnp.int32),
            [pltpu.VMEM((_GCH, _PD), jnp.int32) for _ in range(_GNB)],
            [pltpu.SemaphoreType.DMA for _ in range(_GNB)],
        ],
    )
    return fn(et_pack, idx_flat)


def _agg_body(eh_ref, nb_ref, p_ref, w1_ref, b1_ref, w2_ref, b2_ref,
              aw0_ref, ab0_ref, aw1_ref, ab1_ref, eh2_ref, g_ref):
    eh = eh_ref[...]
    nbs = []
    for k in range(K):
        p = nb_ref[k]
        a = _f16_dec(p & 0xFFFF)
        b = _f16_dec((p >> 16) & 0xFFFF)
        nbs.append(jnp.concatenate([a, b], axis=1))
    kws = []
    for k in range(K):
        pk = p_ref[:, k:k + 1]
        gate = jnp.tanh((2.0 - pk) * eh + pk * nbs[k])
        kws.append(jnp.sum(nbs[k], axis=1, keepdims=True)
                   * jnp.sum(gate, axis=1, keepdims=True))
    kw = jnp.concatenate(kws, axis=1)
    m = jnp.max(kw, axis=1, keepdims=True)
    e = jnp.exp(kw - m)
    sinv = 1.0 / jnp.sum(e, axis=1, keepdims=True)
    enh = (e[:, 0:1] * sinv) * nbs[0]
    for k in range(1, K):
        enh = enh + (e[:, k:k + 1] * sinv) * nbs[k]
    se = lax.dot_general(eh + enh, w1_ref[...], (((1,), (1,)), ((), ())),
                         preferred_element_type=jnp.float32)
    se = _leaky(se + b1_ref[...])
    be = lax.dot_general(eh * enh, w2_ref[...], (((1,), (1,)), ((), ())),
                         preferred_element_type=jnp.float32)
    be = _leaky(be + b2_ref[...])
    eh2 = se + be
    eh2_ref[...] = eh2
    gh = lax.dot_general(eh2, aw0_ref[...], (((1,), (1,)), ((), ())),
                         preferred_element_type=jnp.float32)
    gh = _leaky(gh + ab0_ref[...])
    g = lax.dot_general(gh, aw1_ref[...], (((1,), (1,)), ((), ())),
                        preferred_element_type=jnp.float32)
    g_ref[...] = g + ab1_ref[...]


def _agg(eh, nb, prob, w1, b1, w2, b2, aw0, ab0, aw1, ab1, h):
    return pl.pallas_call(
        _agg_body,
        grid=(HBLK,),
        in_specs=[
            pl.BlockSpec((BLK, D), lambda i, h=h: (i + h * HBLK, 0)),
            pl.BlockSpec((K, BLK, _PD), lambda i: (0, i, 0)),
            pl.BlockSpec((BLK, KP), lambda i: (i, 0)),
            pl.BlockSpec((D, D), lambda i: (0, 0)),
            pl.BlockSpec((1, D), lambda i: (0, 0)),
            pl.BlockSpec((D, D), lambda i: (0, 0)),
            pl.BlockSpec((1, D), lambda i: (0, 0)),
            pl.BlockSpec((256, D), lambda i: (0, 0)),
            pl.BlockSpec((1, 256), lambda i: (0, 0)),
            pl.BlockSpec((128, 256), lambda i: (0, 0)),
            pl.BlockSpec((1, 128), lambda i: (0, 0)),
        ],
        out_specs=[
            pl.BlockSpec((BLK, D), lambda i: (i, 0)),
            pl.BlockSpec((BLK, 128), lambda i: (i, 0)),
        ],
        out_shape=[
            jax.ShapeDtypeStruct((HALF, D), jnp.float32),
            jax.ShapeDtypeStruct((HALF, 128), jnp.float32),
        ],
    )(eh, nb, prob, w1, b1, w2, b2, aw0, ab0, aw1, ab1)


def _read_body(eh2_ref, g_ref, out_ref):
    g = g_ref[:, 0:1]
    m = jnp.max(g)
    e = jnp.exp(g - m)
    w = e / jnp.sum(e)
    out_ref[...] = jnp.sum(w * eh2_ref[...], axis=0, keepdims=True)


def _read(eh2, g):
    return pl.pallas_call(
        _read_body,
        out_shape=jax.ShapeDtypeStruct((1, D), jnp.float32),
    )(eh2, g)


def kernel(x_omic1, x_omic2, x_omic3, x_omic4, x_omic5, x_omic6, x_path,
           sig0_w0, sig0_b0, sig0_w1, sig0_b1,
           sig1_w0, sig1_b0, sig1_w1, sig1_b1,
           sig2_w0, sig2_b0, sig2_w1, sig2_b1,
           sig3_w0, sig3_b0, sig3_w1, sig3_b1,
           sig4_w0, sig4_b0, sig4_w1, sig4_b1,
           sig5_w0, sig5_b0, sig5_w1, sig5_b1,
           fc1_w, fc1_b, wh_w, wh_b, wt_w, wt_b,
           lin1_w, lin1_b, lin2_w, lin2_b,
           att_w0, att_b0, att_w1, att_b1):
    xs = [x_omic1, x_omic2, x_omic3, x_omic4, x_omic5, x_omic6]
    w0s = [sig0_w0, sig1_w0, sig2_w0, sig3_w0, sig4_w0, sig5_w0]
    b0s = [sig0_b0, sig1_b0, sig2_b0, sig3_b0, sig4_b0, sig5_b0]
    w1s = [sig0_w1, sig1_w1, sig2_w1, sig3_w1, sig4_w1, sig5_w1]
    b1s = [sig0_b1, sig1_b1, sig2_b1, sig3_b1, sig4_b1, sig5_b1]
    oargs = []
    for i in range(6):
        oargs += [xs[i][None, :], w0s[i], b0s[i][None, :],
                  w1s[i], b1s[i][None, :]]
    e_omic = _omic(oargs)[:, None, :]

    h, hsum = _fc1(x_path, fc1_w, fc1_b[None, :])
    eh, et, et_pack = _proj(h, hsum, wh_w, wh_b[None, :], wt_w, wt_b[None, :])
    aw1p = jnp.pad(att_w1, ((0, 127), (0, 0)))
    ab1p = jnp.broadcast_to(att_b1[None, :], (1, 128))
    eh2s, gs = [], []
    for hh in range(NSPLIT):
        prob, idx = _topk(eh, et, hh)
        idx_flat = jnp.transpose(idx)[:K].reshape(-1)
        nb = _sc_gather(et_pack, idx_flat).reshape(K, HALF, _PD)
        eh2_h, g_h = _agg(eh, nb, prob,
                          lin1_w, lin1_b[None, :], lin2_w, lin2_b[None, :],
                          att_w0, att_b0[None, :], aw1p, ab1p, hh)
        eh2s.append(eh2_h)
        gs.append(g_h)
    eh2 = jnp.concatenate(eh2s, axis=0)
    g = jnp.concatenate(gs, axis=0)
    e_g = _read(eh2, g)
    return (e_omic, eh2[None], e_g)


# speedup vs baseline: 5.4234x; 1.0002x over previous
import functools

import jax
import jax.numpy as jnp
from jax import lax
from jax.experimental import pallas as pl
from jax.experimental.pallas import tpu as pltpu
from jax.experimental.pallas import tpu_sc as plsc

N = 4096
DIN = 384
D = 512
K = 6
KP = 8
BLK = 256
NBLK = N // BLK
OMIC_PAD = 1536
NEG = -1e30


def _leaky(x):
    return jnp.where(x > 0, x, 0.01 * x)


def _elu(x):
    return jnp.where(x > 0, x, jnp.exp(x) - 1.0)


def _omic_body(*refs):
    o_ref = refs[-1]
    for i in range(6):
        x, w0, b0, w1, b1 = refs[5 * i:5 * i + 5]
        h = lax.dot_general(x[...], w0[...], (((1,), (1,)), ((), ())),
                            preferred_element_type=jnp.float32)
        h = _elu(h + b0[...])
        h = lax.dot_general(h, w1[...], (((1,), (1,)), ((), ())),
                            preferred_element_type=jnp.float32)
        o_ref[i:i + 1, :] = _elu(h + b1[...])


def _omic(args):
    return pl.pallas_call(
        _omic_body,
        out_shape=jax.ShapeDtypeStruct((6, 256), jnp.float32),
    )(*args)


def _fc1_body(xp_ref, w_ref, b_ref, h_ref, s_ref):
    i = pl.program_id(0)
    h = lax.dot_general(xp_ref[...], w_ref[...], (((1,), (1,)), ((), ())),
                        preferred_element_type=jnp.float32)
    h = _leaky(h + b_ref[...])
    h_ref[...] = h
    ps = jnp.sum(h, axis=0, keepdims=True)

    @pl.when(i == 0)
    def _():
        s_ref[...] = ps

    @pl.when(i > 0)
    def _():
        s_ref[...] += ps


def _fc1(x_path, w, b):
    return pl.pallas_call(
        _fc1_body,
        grid=(NBLK,),
        in_specs=[
            pl.BlockSpec((BLK, DIN), lambda i: (i, 0)),
            pl.BlockSpec((D, DIN), lambda i: (0, 0)),
            pl.BlockSpec((1, D), lambda i: (0, 0)),
        ],
        out_specs=[
            pl.BlockSpec((BLK, D), lambda i: (i, 0)),
            pl.BlockSpec((1, D), lambda i: (0, 0)),
        ],
        out_shape=[
            jax.ShapeDtypeStruct((N, D), jnp.float32),
            jax.ShapeDtypeStruct((1, D), jnp.float32),
        ],
    )(x_path, w, b)


_PD = D // 2


def _f16_enc(x):
    y = lax.bitcast_convert_type(x * jnp.float32(2.0 ** -112), jnp.int32)
    y = y + 0xFFF + ((y >> 13) & 1)
    return ((y >> 16) & 0x8000) | ((y >> 13) & 0x7FFF)


def _f16_dec(h):
    z = ((h & 0x8000) << 16) | ((h & 0x7FFF) << 13)
    return lax.bitcast_convert_type(z, jnp.float32) * jnp.float32(2.0 ** 112)


def _proj_body(h_ref, s_ref, wh_ref, bh_ref, wt_ref, bt_ref, eh_ref, et_ref,
               pk_ref):
    x = (h_ref[...] + s_ref[...] * (1.0 / N)) * 0.5
    eh = lax.dot_general(x, wh_ref[...], (((1,), (1,)), ((), ())),
                         preferred_element_type=jnp.float32)
    eh_ref[...] = eh + bh_ref[...]
    et = lax.dot_general(x, wt_ref[...], (((1,), (1,)), ((), ())),
                         preferred_element_type=jnp.float32)
    et = et + bt_ref[...]
    et_ref[...] = et
    lo = _f16_enc(et[:, :_PD])
    hi = _f16_enc(et[:, _PD:])
    pk_ref[...] = lo | (hi << 16)


def _proj(h, s, wh, bh, wt, bt):
    return pl.pallas_call(
        _proj_body,
        grid=(NBLK,),
        in_specs=[
            pl.BlockSpec((BLK, D), lambda i: (i, 0)),
            pl.BlockSpec((1, D), lambda i: (0, 0)),
            pl.BlockSpec((D, D), lambda i: (0, 0)),
            pl.BlockSpec((1, D), lambda i: (0, 0)),
            pl.BlockSpec((D, D), lambda i: (0, 0)),
            pl.BlockSpec((1, D), lambda i: (0, 0)),
        ],
        out_specs=[
            pl.BlockSpec((BLK, D), lambda i: (i, 0)),
            pl.BlockSpec((BLK, D), lambda i: (i, 0)),
            pl.BlockSpec((BLK, _PD), lambda i: (i, 0)),
        ],
        out_shape=[
            jax.ShapeDtypeStruct((N, D), jnp.float32),
            jax.ShapeDtypeStruct((N, D), jnp.float32),
            jax.ShapeDtypeStruct((N, _PD), jnp.int32),
        ],
    )(h, s, wh, bh, wt, bt)


def _topk_body(eh_ref, et_ref, prob_ref, idx_ref):
    scale = D ** -0.5
    s = lax.dot_general(eh_ref[...] * scale, et_ref[...],
                        (((1,), (1,)), ((), ())),
                        preferred_element_type=jnp.float32)
    colid = lax.broadcasted_iota(jnp.int32, (BLK, N), 1)
    vals = s
    vcols = []
    icols = []
    for j in range(K):
        m = jnp.max(vals, axis=1, keepdims=True)
        sel = vals >= m
        idx = jnp.min(jnp.where(sel, colid, jnp.int32(2 ** 30)),
                      axis=1, keepdims=True)
        vcols.append(m)
        icols.append(idx)
        if j + 1 < K:
            vals = jnp.where(colid == idx, NEG, vals)
    v6 = jnp.concatenate(vcols, axis=1)
    mm = vcols[0]
    e6 = jnp.exp(v6 - mm)
    p6 = e6 / jnp.sum(e6, axis=1, keepdims=True)
    z1 = jnp.zeros((BLK, 1), jnp.float32)
    prob_ref[...] = jnp.concatenate([p6, z1, z1], axis=1)
    zi = jnp.zeros((BLK, 1), jnp.int32)
    idx_ref[...] = jnp.concatenate(icols + [zi, zi], axis=1)


NSPLIT = 4
HALF = N // NSPLIT
HBLK = HALF // BLK


def _topk(eh, et, h):
    return pl.pallas_call(
        _topk_body,
        grid=(HBLK,),
        in_specs=[
            pl.BlockSpec((BLK, D), lambda i, h=h: (i + h * HBLK, 0)),
            pl.BlockSpec((N, D), lambda i: (0, 0)),
        ],
        out_specs=[
            pl.BlockSpec((BLK, KP), lambda i: (i, 0)),
            pl.BlockSpec((BLK, KP), lambda i: (i, 0)),
        ],
        out_shape=[
            jax.ShapeDtypeStruct((HALF, KP), jnp.float32),
            jax.ShapeDtypeStruct((HALF, KP), jnp.int32),
        ],
    )(eh, et)


_GROWS = K * HALF
_GPER = _GROWS // 32
_GNB = 2
_GCH = 96
_GNCH = _GPER // _GCH


def _sc_gather_body(et_hbm, idx_hbm, out_hbm, idx_v, bufs, sems):
    wid = lax.axis_index("s") * 2 + lax.axis_index("c")
    base = wid * _GPER
    pltpu.sync_copy(idx_hbm.at[pl.ds(base, _GPER)], idx_v)
    cps = [None] * _GNCH
    for c in range(_GNB):
        cps[c] = pltpu.async_copy(
            et_hbm.at[idx_v.at[pl.ds(c * _GCH, _GCH)]], bufs[c], sems[c])
    for c in range(_GNCH):
        b = c % _GNB
        cps[c].wait()
        pltpu.sync_copy(bufs[b], out_hbm.at[pl.ds(base + c * _GCH, _GCH)])
        nxt = c + _GNB
        if nxt < _GNCH:
            cps[nxt] = pltpu.async_copy(
                et_hbm.at[idx_v.at[pl.ds(nxt * _GCH, _GCH)]], bufs[b],
                sems[b])


def _sc_gather(et_pack, idx_flat):
    mesh = plsc.VectorSubcoreMesh(core_axis_name="c", subcore_axis_name="s")
    fn = pl.kernel(
        _sc_gather_body,
        out_type=jax.ShapeDtypeStruct((_GROWS, _PD), jnp.int32),
        mesh=mesh,
        scratch_types=[
            pltpu.VMEM((_GPER,), jnp.int32),
            [pltpu.VMEM((_GCH, _PD), jnp.int32) for _ in range(_GNB)],
            [pltpu.SemaphoreType.DMA for _ in range(_GNB)],
        ],
    )
    return fn(et_pack, idx_flat)


def _agg_body(eh_ref, nb_ref, p_ref, w1_ref, b1_ref, w2_ref, b2_ref,
              aw0_ref, ab0_ref, aw1_ref, ab1_ref, eh2_ref, g_ref):
    eh = eh_ref[...]
    nbs = []
    for k in range(K):
        p = nb_ref[k]
        a = _f16_dec(p & 0xFFFF)
        b = _f16_dec((p >> 16) & 0xFFFF)
        nbs.append(jnp.concatenate([a, b], axis=1))
    kws = []
    for k in range(K):
        pk = p_ref[:, k:k + 1]
        gate = jnp.tanh((2.0 - pk) * eh + pk * nbs[k])
        kws.append(jnp.sum(nbs[k], axis=1, keepdims=True)
                   * jnp.sum(gate, axis=1, keepdims=True))
    kw = jnp.concatenate(kws, axis=1)
    m = jnp.max(kw, axis=1, keepdims=True)
    e = jnp.exp(kw - m)
    sinv = 1.0 / jnp.sum(e, axis=1, keepdims=True)
    enh = (e[:, 0:1] * sinv) * nbs[0]
    for k in range(1, K):
        enh = enh + (e[:, k:k + 1] * sinv) * nbs[k]
    se = lax.dot_general(eh + enh, w1_ref[...], (((1,), (1,)), ((), ())),
                         preferred_element_type=jnp.float32)
    se = _leaky(se + b1_ref[...])
    be = lax.dot_general(eh * enh, w2_ref[...], (((1,), (1,)), ((), ())),
                         preferred_element_type=jnp.float32)
    be = _leaky(be + b2_ref[...])
    eh2 = se + be
    eh2_ref[...] = eh2
    gh = lax.dot_general(eh2, aw0_ref[...], (((1,), (1,)), ((), ())),
                         preferred_element_type=jnp.float32)
    gh = _leaky(gh + ab0_ref[...])
    g = lax.dot_general(gh, aw1_ref[...], (((1,), (1,)), ((), ())),
                        preferred_element_type=jnp.float32)
    g_ref[...] = g + ab1_ref[...]


def _agg(eh, nb, prob, w1, b1, w2, b2, aw0, ab0, aw1, ab1, h):
    return pl.pallas_call(
        _agg_body,
        grid=(HBLK,),
        in_specs=[
            pl.BlockSpec((BLK, D), lambda i, h=h: (i + h * HBLK, 0)),
            pl.BlockSpec((K, BLK, _PD), lambda i: (0, i, 0)),
            pl.BlockSpec((BLK, KP), lambda i: (i, 0)),
            pl.BlockSpec((D, D), lambda i: (0, 0)),
            pl.BlockSpec((1, D), lambda i: (0, 0)),
            pl.BlockSpec((D, D), lambda i: (0, 0)),
            pl.BlockSpec((1, D), lambda i: (0, 0)),
            pl.BlockSpec((256, D), lambda i: (0, 0)),
            pl.BlockSpec((1, 256), lambda i: (0, 0)),
            pl.BlockSpec((128, 256), lambda i: (0, 0)),
            pl.BlockSpec((1, 128), lambda i: (0, 0)),
        ],
        out_specs=[
            pl.BlockSpec((BLK, D), lambda i: (i, 0)),
            pl.BlockSpec((BLK, 128), lambda i: (i, 0)),
        ],
        out_shape=[
            jax.ShapeDtypeStruct((HALF, D), jnp.float32),
            jax.ShapeDtypeStruct((HALF, 128), jnp.float32),
        ],
    )(eh, nb, prob, w1, b1, w2, b2, aw0, ab0, aw1, ab1)


def _read_body(eh2_ref, g_ref, out_ref):
    g = g_ref[:, 0:1]
    m = jnp.max(g)
    e = jnp.exp(g - m)
    w = e / jnp.sum(e)
    out_ref[...] = jnp.sum(w * eh2_ref[...], axis=0, keepdims=True)


def _read(eh2, g):
    return pl.pallas_call(
        _read_body,
        out_shape=jax.ShapeDtypeStruct((1, D), jnp.float32),
    )(eh2, g)


def kernel(x_omic1, x_omic2, x_omic3, x_omic4, x_omic5, x_omic6, x_path,
           sig0_w0, sig0_b0, sig0_w1, sig0_b1,
           sig1_w0, sig1_b0, sig1_w1, sig1_b1,
           sig2_w0, sig2_b0, sig2_w1, sig2_b1,
           sig3_w0, sig3_b0, sig3_w1, sig3_b1,
           sig4_w0, sig4_b0, sig4_w1, sig4_b1,
           sig5_w0, sig5_b0, sig5_w1, sig5_b1,
           fc1_w, fc1_b, wh_w, wh_b, wt_w, wt_b,
           lin1_w, lin1_b, lin2_w, lin2_b,
           att_w0, att_b0, att_w1, att_b1):
    xs = [x_omic1, x_omic2, x_omic3, x_omic4, x_omic5, x_omic6]
    w0s = [sig0_w0, sig1_w0, sig2_w0, sig3_w0, sig4_w0, sig5_w0]
    b0s = [sig0_b0, sig1_b0, sig2_b0, sig3_b0, sig4_b0, sig5_b0]
    w1s = [sig0_w1, sig1_w1, sig2_w1, sig3_w1, sig4_w1, sig5_w1]
    b1s = [sig0_b1, sig1_b1, sig2_b1, sig3_b1, sig4_b1, sig5_b1]
    oargs = []
    for i in range(6):
        oargs += [xs[i][None, :], w0s[i], b0s[i][None, :],
                  w1s[i], b1s[i][None, :]]
    e_omic = _omic(oargs)[:, None, :]

    h, hsum = _fc1(x_path, fc1_w, fc1_b[None, :])
    eh, et, et_pack = _proj(h, hsum, wh_w, wh_b[None, :], wt_w, wt_b[None, :])
    aw1p = jnp.pad(att_w1, ((0, 127), (0, 0)))
    ab1p = jnp.broadcast_to(att_b1[None, :], (1, 128))
    eh2s, gs = [], []
    for hh in range(NSPLIT):
        prob, idx = _topk(eh, et, hh)
        idx_flat = jnp.transpose(idx)[:K].reshape(-1)
        nb = _sc_gather(et_pack, idx_flat).reshape(K, HALF, _PD)
        eh2_h, g_h = _agg(eh, nb, prob,
                          lin1_w, lin1_b[None, :], lin2_w, lin2_b[None, :],
                          att_w0, att_b0[None, :], aw1p, ab1p, hh)
        eh2s.append(eh2_h)
        gs.append(g_h)
    eh2 = jnp.concatenate(eh2s, axis=0)
    g = jnp.concatenate(gs, axis=0)
    e_g = _read(eh2, g)
    return (e_omic, eh2[None], e_g)
